# baseline (speedup 1.0000x reference)
import functools

import jax
import jax.numpy as jnp
from jax import lax
from jax.experimental import pallas as pl
from jax.experimental.pallas import tpu as pltpu
from jax.experimental.pallas import tpu_sc as plsc

F32 = jnp.float32
I32 = jnp.int32

_NC = 2
_NS = 16
_NW = _NC * _NS
_B = 128


def _silu(x):
    return x * jax.nn.sigmoid(x)


def _pre_body(f_ref, ws_ref, wd_ref, eb_ref, ps_ref, pd_ref):
    f = f_ref[...]
    ps_ref[...] = jnp.dot(f, ws_ref[...], preferred_element_type=F32)
    pd_ref[...] = jnp.dot(f, wd_ref[...], preferred_element_type=F32) + eb_ref[...]


def _pre_call(f, ws, wd, eb):
    n = f.shape[0]
    m = ws.shape[1]
    return pl.pallas_call(
        _pre_body,
        out_shape=(jax.ShapeDtypeStruct((n, m), F32),
                   jax.ShapeDtypeStruct((n, m), F32)),
    )(f, ws, wd, eb)


def _edge1_body(g_ref, rd_ref, wdist_ref, ew2_ref, eb2_ref, cw1_ref, cb1_ref,
                cw2_ref, cb2_ref, mc_ref, rcw_ref):
    g = g_ref[...]
    rd = rd_ref[...]
    dist = rd[:, 3:4]
    h1 = _silu(g + dist * wdist_ref[...])
    m = _silu(jnp.dot(h1, ew2_ref[...], preferred_element_type=F32) + eb2_ref[...])
    t = _silu(jnp.dot(m, cw1_ref[...], preferred_element_type=F32) + cb1_ref[...])
    cw = jnp.dot(t, cw2_ref[...], preferred_element_type=F32)[:, 0:1] + cb2_ref[...]
    mask = (lax.broadcasted_iota(I32, rd.shape, 1) < 3).astype(F32)
    mc_ref[...] = m
    rcw_ref[...] = rd * cw * mask


def _edge1_call(g, rd, wdist, ew2, eb2, cw1, cb1, cw2p, cb2, bt):
    e = g.shape[0]
    mdim = g.shape[1]
    grid = e // bt
    full = lambda i: (0, 0)
    return pl.pallas_call(
        _edge1_body,
        grid=(grid,),
        in_specs=[
            pl.BlockSpec((bt, mdim), lambda i: (i, 0)),
            pl.BlockSpec((bt, 8), lambda i: (i, 0)),
            pl.BlockSpec(wdist.shape, full),
            pl.BlockSpec(ew2.shape, full),
            pl.BlockSpec(eb2.shape, full),
            pl.BlockSpec(cw1.shape, full),
            pl.BlockSpec(cb1.shape, full),
            pl.BlockSpec(cw2p.shape, full),
            pl.BlockSpec(cb2.shape, full),
        ],
        out_specs=(pl.BlockSpec((bt, mdim), lambda i: (i, 0)),
                   pl.BlockSpec((bt, 8), lambda i: (i, 0))),
        out_shape=(jax.ShapeDtypeStruct((e, mdim), F32),
                   jax.ShapeDtypeStruct((e, 8), F32)),
    )(g, rd, wdist, ew2, eb2, cw1, cb1, cw2p, cb2)


def _edge2_body(g_ref, rd_ref, wdist_ref, ew2_ref, eb2_ref, m_ref):
    g = g_ref[...]
    dist = rd_ref[...][:, 3:4]
    h1 = _silu(g + dist * wdist_ref[...])
    m_ref[...] = _silu(jnp.dot(h1, ew2_ref[...], preferred_element_type=F32)
                       + eb2_ref[...])


def _edge2_call(g, rd, wdist, ew2, eb2, bt):
    e = g.shape[0]
    mdim = g.shape[1]
    grid = e // bt
    full = lambda i: (0, 0)
    return pl.pallas_call(
        _edge2_body,
        grid=(grid,),
        in_specs=[
            pl.BlockSpec((bt, mdim), lambda i: (i, 0)),
            pl.BlockSpec((bt, 8), lambda i: (i, 0)),
            pl.BlockSpec(wdist.shape, full),
            pl.BlockSpec(ew2.shape, full),
            pl.BlockSpec(eb2.shape, full),
        ],
        out_specs=pl.BlockSpec((bt, mdim), lambda i: (i, 0)),
        out_shape=jax.ShapeDtypeStruct((e, mdim), F32),
    )(g, rd, wdist, ew2, eb2)


def _node1_body(f_ref, c4_ref, aggp_ref, csp_ref, nw1a_ref, nw1b_ref, nb1_ref,
                nw2_ref, nb2_ref, fcw_ref, fcb_ref, ws2_ref, wd2_ref, eb2l_ref,
                f2_ref, c24_ref, ps2_ref, pd2_ref):
    f = f_ref[...]
    ap = aggp_ref[...]
    agg = ap[0]
    for i in range(1, ap.shape[0]):
        agg = agg + ap[i]
    h = _silu(jnp.dot(f, nw1a_ref[...], preferred_element_type=F32)
              + jnp.dot(agg, nw1b_ref[...], preferred_element_type=F32)
              + nb1_ref[...])
    f1 = f + jnp.dot(h, nw2_ref[...], preferred_element_type=F32) + nb2_ref[...]
    f2 = jnp.dot(f1, fcw_ref[...], preferred_element_type=F32) + fcb_ref[...]
    f2_ref[...] = f2
    cp = csp_ref[...]
    cs = cp[:, 0:4]
    for i in range(1, cp.shape[1] // 4):
        cs = cs + cp[:, 4 * i:4 * i + 4]
    c24_ref[...] = c4_ref[...] + cs
    ps2_ref[...] = jnp.dot(f2, ws2_ref[...], preferred_element_type=F32)
    pd2_ref[...] = jnp.dot(f2, wd2_ref[...], preferred_element_type=F32) + eb2l_ref[...]


def _node1_call(f, c4, aggp, csp, nw1a, nw1b, nb1, nw2, nb2, fcw, fcb,
                ws2, wd2, eb2l):
    n, d = f.shape
    lat = fcw.shape[1]
    m = ws2.shape[1]
    np_ = aggp.shape[0]
    bt2 = 2000
    grid = n // bt2
    full = lambda i: (0, 0)
    return pl.pallas_call(
        _node1_body,
        grid=(grid,),
        in_specs=[
            pl.BlockSpec((bt2, d), lambda i: (i, 0)),
            pl.BlockSpec((bt2, 4), lambda i: (i, 0)),
            pl.BlockSpec((np_, bt2, m), lambda i: (0, i, 0)),
            pl.BlockSpec((bt2, csp.shape[1]), lambda i: (i, 0)),
            pl.BlockSpec(nw1a.shape, full),
            pl.BlockSpec(nw1b.shape, full),
            pl.BlockSpec(nb1.shape, full),
            pl.BlockSpec(nw2.shape, full),
            pl.BlockSpec(nb2.shape, full),
            pl.BlockSpec(fcw.shape, full),
            pl.BlockSpec(fcb.shape, full),
            pl.BlockSpec(ws2.shape, full),
            pl.BlockSpec(wd2.shape, full),
            pl.BlockSpec(eb2l.shape, full),
        ],
        out_specs=(pl.BlockSpec((bt2, lat), lambda i: (i, 0)),
                   pl.BlockSpec((bt2, 4), lambda i: (i, 0)),
                   pl.BlockSpec((bt2, m), lambda i: (i, 0)),
                   pl.BlockSpec((bt2, m), lambda i: (i, 0))),
        out_shape=(jax.ShapeDtypeStruct((n, lat), F32),
                   jax.ShapeDtypeStruct((n, 4), F32),
                   jax.ShapeDtypeStruct((n, m), F32),
                   jax.ShapeDtypeStruct((n, m), F32)),
    )(f, c4, aggp, csp, nw1a, nw1b, nb1, nw2, nb2, fcw, fcb, ws2, wd2, eb2l)


def _node2_body(f2_ref, aggp_ref, nw1a_ref, nw1b_ref, nb1_ref, nw2_ref,
                nb2_ref, fc1w_ref, fc1b_ref, out_ref):
    f2 = f2_ref[...]
    ap = aggp_ref[...]
    agg = ap[0]
    for i in range(1, ap.shape[0]):
        agg = agg + ap[i]
    h = _silu(jnp.dot(f2, nw1a_ref[...], preferred_element_type=F32)
              + jnp.dot(agg, nw1b_ref[...], preferred_element_type=F32)
              + nb1_ref[...])
    f3 = f2 + jnp.dot(h, nw2_ref[...], preferred_element_type=F32) + nb2_ref[...]
    out_ref[...] = jnp.dot(f3, fc1w_ref[...], preferred_element_type=F32) + fc1b_ref[...]


def _node2_call(f2, aggp, nw1a, nw1b, nb1, nw2, nb2, fc1w, fc1b):
    n, lat = f2.shape
    out = fc1w.shape[1]
    m = nw1b.shape[0]
    np_ = aggp.shape[0]
    bt2 = 2000
    grid = n // bt2
    full = lambda i: (0, 0)
    return pl.pallas_call(
        _node2_body,
        grid=(grid,),
        in_specs=[
            pl.BlockSpec((bt2, lat), lambda i: (i, 0)),
            pl.BlockSpec((np_, bt2, m), lambda i: (0, i, 0)),
            pl.BlockSpec(nw1a.shape, full),
            pl.BlockSpec(nw1b.shape, full),
            pl.BlockSpec(nb1.shape, full),
            pl.BlockSpec(nw2.shape, full),
            pl.BlockSpec(nb2.shape, full),
            pl.BlockSpec(fc1w.shape, full),
            pl.BlockSpec(fc1b.shape, full),
        ],
        out_specs=pl.BlockSpec((bt2, out), lambda i: (i, 0)),
        out_shape=jax.ShapeDtypeStruct((n, out), F32),
    )(f2, aggp, nw1a, nw1b, nb1, nw2, nb2, fc1w, fc1b)


def _sc_gather(ps, pd, c4flat, src, dst):
    n, m = ps.shape
    e = src.shape[0]
    nblk = e // _B
    mesh = plsc.VectorSubcoreMesh(core_axis_name="c", subcore_axis_name="s")
    grp = m // 16
    clen = c4flat.shape[0]

    @functools.partial(
        pl.kernel,
        out_type=(jax.ShapeDtypeStruct((e, m), F32),
                  jax.ShapeDtypeStruct((e * 8,), F32)),
        mesh=mesh,
        compiler_params=pltpu.CompilerParams(needs_layout_passes=False),
        scratch_types=[
            pltpu.VMEM((_B,), I32),
            pltpu.VMEM((_B,), I32),
            pltpu.VMEM((_B,), I32),
            pltpu.VMEM((_B,), I32),
            pltpu.VMEM((_B, m), F32),
            pltpu.VMEM((_B, m), F32),
            pltpu.VMEM((_B, m), F32),
            pltpu.VMEM((_B, m), F32),
            pltpu.VMEM((_B * 8,), F32),
            pltpu.VMEM((_B * 8,), F32),
            pltpu.VMEM((clen,), F32),
            pltpu.SemaphoreType.DMA,
            pltpu.SemaphoreType.DMA,
            pltpu.SemaphoreType.DMA,
            pltpu.SemaphoreType.DMA,
            pltpu.SemaphoreType.DMA,
            pltpu.SemaphoreType.DMA,
        ],
    )
    def gather_k(ps_hbm, pd_hbm, c4_hbm, src_hbm, dst_hbm, g_hbm, rd_hbm,
                 is0, id0, is1, id1, ba0, bb0, ba1, bb1, rb0, rb1, ctab,
                 sa0, sb0, sa1, sb1, sw0, sw1):
        wid = lax.axis_index("s") * _NC + lax.axis_index("c")
        pltpu.sync_copy(c4_hbm, ctab)
        zeros16 = jnp.zeros((16,), F32)

        def zbody(k, _):
            rb0[pl.ds(k * 16, 16)] = zeros16
            rb1[pl.ds(k * 16, 16)] = zeros16
            return 0
        lax.fori_loop(0, (_B * 8) // 16, zbody, 0)

        nmine = (nblk - 1 - wid) // _NW + 1
        npair = (nmine + 1) // 2
        iota16 = lax.iota(I32, 16)

        def reldist(isx, idxx, rb):
            for j in range(_B // 16):
                s16 = isx[pl.ds(j * 16, 16)] * 4
                d16 = idxx[pl.ds(j * 16, 16)] * 4
                xs = plsc.load_gather(ctab, [s16])
                ys = plsc.load_gather(ctab, [s16 + 1])
                zs = plsc.load_gather(ctab, [s16 + 2])
                xd = plsc.load_gather(ctab, [d16])
                yd = plsc.load_gather(ctab, [d16 + 1])
                zd = plsc.load_gather(ctab, [d16 + 2])
                rx = xs - xd
                ry = ys - yd
                rz = zs - zd
                dist = rx * rx + ry * ry + rz * rz
                off = (j * 16 + iota16) * 8
                plsc.store_scatter(rb, [off], rx)
                plsc.store_scatter(rb, [off + 1], ry)
                plsc.store_scatter(rb, [off + 2], rz)
                plsc.store_scatter(rb, [off + 3], dist)

        def addrows(ba, bb):
            def addrow(r, _):
                for k in range(grp):
                    sl = pl.ds(k * 16, 16)
                    plsc.addupdate(ba.at[r, sl], bb[r, sl])
                return 0
            lax.fori_loop(0, _B, addrow, 0)

        def pair_body(it, _):
            ja = it * 2
            jb = ja + 1
            hasb = jb < nmine
            basea = (wid + ja * _NW) * _B

            @pl.when(it > 0)
            def _():
                pltpu.make_async_copy(g_hbm.at[pl.ds(0, _B)], ba0, sw0).wait()
                pltpu.make_async_copy(rd_hbm.at[pl.ds(0, _B * 8)], rb0, sw0).wait()
                pltpu.make_async_copy(g_hbm.at[pl.ds(0, _B)], ba1, sw1).wait()
                pltpu.make_async_copy(rd_hbm.at[pl.ds(0, _B * 8)], rb1, sw1).wait()

            pltpu.sync_copy(src_hbm.at[pl.ds(basea, _B)], is0)
            pltpu.sync_copy(dst_hbm.at[pl.ds(basea, _B)], id0)
            cpa = pltpu.async_copy(ps_hbm.at[is0], ba0, sa0)
            cpb = pltpu.async_copy(pd_hbm.at[id0], bb0, sb0)

            @pl.when(hasb)
            def _():
                baseb = (wid + jb * _NW) * _B
                pltpu.sync_copy(src_hbm.at[pl.ds(baseb, _B)], is1)
                pltpu.sync_copy(dst_hbm.at[pl.ds(baseb, _B)], id1)
                pltpu.async_copy(ps_hbm.at[is1], ba1, sa1)
                pltpu.async_copy(pd_hbm.at[id1], bb1, sb1)

            reldist(is0, id0, rb0)
            cpa.wait()
            cpb.wait()
            addrows(ba0, bb0)
            pltpu.async_copy(ba0, g_hbm.at[pl.ds(basea, _B)], sw0)
            pltpu.async_copy(rb0, rd_hbm.at[pl.ds(basea * 8, _B * 8)], sw0)

            @pl.when(hasb)
            def _():
                baseb = (wid + jb * _NW) * _B
                reldist(is1, id1, rb1)
                pltpu.make_async_copy(ps_hbm.at[pl.ds(0, _B)], ba1, sa1).wait()
                pltpu.make_async_copy(pd_hbm.at[pl.ds(0, _B)], bb1, sb1).wait()
                addrows(ba1, bb1)
                pltpu.async_copy(ba1, g_hbm.at[pl.ds(baseb, _B)], sw1)
                pltpu.async_copy(rb1, rd_hbm.at[pl.ds(baseb * 8, _B * 8)], sw1)
            return 0

        lax.fori_loop(0, npair, pair_body, 0)

        pltpu.make_async_copy(g_hbm.at[pl.ds(0, _B)], ba0, sw0).wait()
        pltpu.make_async_copy(rd_hbm.at[pl.ds(0, _B * 8)], rb0, sw0).wait()

        @pl.when((nmine % 2) == 0)
        def _():
            pltpu.make_async_copy(g_hbm.at[pl.ds(0, _B)], ba1, sw1).wait()
            pltpu.make_async_copy(rd_hbm.at[pl.ds(0, _B * 8)], rb1, sw1).wait()

    return gather_k(ps, pd, c4flat, src, dst)


def _sc_scatter_call(mrows, dst, n):
    e, mdim = mrows.shape
    nblk = e // _B
    zrows = 16
    npad = -(-n // (_NS * zrows)) * (_NS * zrows)
    rows_per_tile = npad // _NS
    nz = rows_per_tile // zrows
    mesh = plsc.VectorSubcoreMesh(core_axis_name="c", subcore_axis_name="s")
    grp = mdim // 16

    @functools.partial(
        pl.kernel,
        out_type=jax.ShapeDtypeStruct((2 * npad, mdim), F32),
        mesh=mesh,
        compiler_params=pltpu.CompilerParams(needs_layout_passes=False),
        scratch_types=[
            pltpu.VMEM((_B,), I32),
            pltpu.VMEM((_B,), I32),
            pltpu.VMEM((_B, mdim), F32),
            pltpu.VMEM((_B, mdim), F32),
            pltpu.VMEM((zrows, mdim), F32),
            pltpu.VMEM_SHARED((npad, mdim), F32),
            pltpu.SemaphoreType.DMA,
            pltpu.SemaphoreType.DMA,
            pltpu.SemaphoreType.DMA,
            pltpu.SemaphoreType.DMA,
        ],
    )
    def body(m_hbm, dst_hbm, aggp_hbm, ix0, ix1, mb0, mb1, zbuf, agg_sh,
             sl0, sl1, ss0, ss1):
        cid = lax.axis_index("c")
        sid = lax.axis_index("s")
        wid = sid * _NC + cid
        zeros16 = jnp.zeros((16,), F32)

        def zrow(r, _):
            for k in range(grp):
                zbuf[r, pl.ds(k * 16, 16)] = zeros16
            return 0
        lax.fori_loop(0, zrows, zrow, 0)

        base_row = sid * rows_per_tile
        for k in range(nz):
            pltpu.sync_copy(zbuf, agg_sh.at[pl.ds(base_row + k * zrows, zrows)])
        plsc.subcore_barrier()

        nmine = (nblk - 1 - wid) // _NW + 1
        npair = (nmine + 1) // 2

        def pair_body(it, _):
            ja = it * 2
            jb = ja + 1
            hasb = jb < nmine
            basea = (wid + ja * _NW) * _B

            @pl.when(it > 0)
            def _():
                pltpu.make_async_copy(m_hbm.at[pl.ds(0, _B)], mb0, ss0).wait()
                pltpu.make_async_copy(m_hbm.at[pl.ds(0, _B)], mb1, ss1).wait()

            cpi = pltpu.async_copy(dst_hbm.at[pl.ds(basea, _B)], ix0, sl0)
            cpm = pltpu.async_copy(m_hbm.at[pl.ds(basea, _B)], mb0, sl0)

            @pl.when(hasb)
            def _():
                baseb = (wid + jb * _NW) * _B
                pltpu.async_copy(dst_hbm.at[pl.ds(baseb, _B)], ix1, sl1)
                pltpu.async_copy(m_hbm.at[pl.ds(baseb, _B)], mb1, sl1)

            cpi.wait()
            cpm.wait()
            pltpu.async_copy(mb0, agg_sh.at[ix0], ss0, add=True)

            @pl.when(hasb)
            def _():
                pltpu.make_async_copy(dst_hbm.at[pl.ds(0, _B)], ix1, sl1).wait()
                pltpu.make_async_copy(m_hbm.at[pl.ds(0, _B)], mb1, sl1).wait()
                pltpu.async_copy(mb1, agg_sh.at[ix1], ss1, add=True)
            return 0

        lax.fori_loop(0, npair, pair_body, 0)

        pltpu.make_async_copy(m_hbm.at[pl.ds(0, _B)], mb0, ss0).wait()

        @pl.when((nmine % 2) == 0)
        def _():
            pltpu.make_async_copy(m_hbm.at[pl.ds(0, _B)], mb1, ss1).wait()
        plsc.subcore_barrier()

        out_base = cid * npad + base_row
        pltpu.sync_copy(agg_sh.at[pl.ds(base_row, rows_per_tile)],
                        aggp_hbm.at[pl.ds(out_base, rows_per_tile)])

    return body(mrows, dst)


def _sc_cscatter_call(rcwflat, dst, n):
    e = dst.shape[0]
    nblk = e // _B
    zrows = 16
    npad = -(-n // (_NS * zrows)) * (_NS * zrows)
    rows_per_tile = npad // _NS
    nz = rows_per_tile // zrows
    mesh = plsc.VectorSubcoreMesh(core_axis_name="c", subcore_axis_name="s")

    @functools.partial(
        pl.kernel,
        out_type=jax.ShapeDtypeStruct((2 * npad, 128), F32),
        mesh=mesh,
        compiler_params=pltpu.CompilerParams(needs_layout_passes=False),
        scratch_types=[
            pltpu.VMEM((_B,), I32),
            pltpu.VMEM((_B,), I32),
            pltpu.VMEM((_B * 8,), F32),
            pltpu.VMEM((_B * 8,), F32),
            pltpu.VMEM((_B, 128), F32),
            pltpu.VMEM((_B, 128), F32),
            pltpu.VMEM((zrows, 128), F32),
            pltpu.VMEM_SHARED((npad, 128), F32),
            pltpu.SemaphoreType.DMA,
            pltpu.SemaphoreType.DMA,
            pltpu.SemaphoreType.DMA,
            pltpu.SemaphoreType.DMA,
        ],
    )
    def body(rcw_hbm, dst_hbm, csp_hbm, ix0, ix1, rb0, rb1, eb0, eb1,
             zbuf, cs_sh, sl0, sl1, ss0, ss1):
        cid = lax.axis_index("c")
        sid = lax.axis_index("s")
        wid = sid * _NC + cid
        zeros16 = jnp.zeros((16,), F32)
        iota16 = lax.iota(I32, 16)

        def zrow(r, _):
            for k in range(8):
                zbuf[r, pl.ds(k * 16, 16)] = zeros16
            return 0
        lax.fori_loop(0, zrows, zrow, 0)

        def zerow(r, _):
            for k in range(8):
                eb0[r, pl.ds(k * 16, 16)] = zeros16
                eb1[r, pl.ds(k * 16, 16)] = zeros16
            return 0
        lax.fori_loop(0, _B, zerow, 0)
        base_row = sid * rows_per_tile
        for k in range(nz):
            pltpu.sync_copy(zbuf, cs_sh.at[pl.ds(base_row + k * zrows, zrows)])
        plsc.subcore_barrier()

        nmine = (nblk - 1 - wid) // _NW + 1
        npair = (nmine + 1) // 2

        def expand(rb, eb):
            for g in range(_B // 16):
                rows = g * 16 + iota16
                for c in range(3):
                    v = plsc.load_gather(rb, [rows * 8 + c])
                    plsc.store_scatter(eb, [rows, jnp.zeros((16,), I32) + c], v)

        def pair_body(it, _):
            ja = it * 2
            jb = ja + 1
            hasb = jb < nmine
            basea = (wid + ja * _NW) * _B

            @pl.when(it > 0)
            def _():
                pltpu.make_async_copy(csp_hbm.at[pl.ds(0, _B)], eb0, ss0).wait()
                pltpu.make_async_copy(csp_hbm.at[pl.ds(0, _B)], eb1, ss1).wait()

            cpi = pltpu.async_copy(dst_hbm.at[pl.ds(basea, _B)], ix0, sl0)
            cpr = pltpu.async_copy(rcw_hbm.at[pl.ds(basea * 8, _B * 8)], rb0, sl0)

            @pl.when(hasb)
            def _():
                baseb = (wid + jb * _NW) * _B
                pltpu.async_copy(dst_hbm.at[pl.ds(baseb, _B)], ix1, sl1)
                pltpu.async_copy(rcw_hbm.at[pl.ds(baseb * 8, _B * 8)], rb1, sl1)

            cpi.wait()
            cpr.wait()
            expand(rb0, eb0)
            pltpu.async_copy(eb0, cs_sh.at[ix0], ss0, add=True)

            @pl.when(hasb)
            def _():
                pltpu.make_async_copy(dst_hbm.at[pl.ds(0, _B)], ix1, sl1).wait()
                pltpu.make_async_copy(rcw_hbm.at[pl.ds(0, _B * 8)], rb1, sl1).wait()
                expand(rb1, eb1)
                pltpu.async_copy(eb1, cs_sh.at[ix1], ss1, add=True)
            return 0

        lax.fori_loop(0, npair, pair_body, 0)

        pltpu.make_async_copy(csp_hbm.at[pl.ds(0, _B)], eb0, ss0).wait()

        @pl.when((nmine % 2) == 0)
        def _():
            pltpu.make_async_copy(csp_hbm.at[pl.ds(0, _B)], eb1, ss1).wait()
        plsc.subcore_barrier()

        out_base = cid * npad + base_row
        pltpu.sync_copy(cs_sh.at[pl.ds(base_row, rows_per_tile)],
                        csp_hbm.at[pl.ds(out_base, rows_per_tile)])

    return body(rcwflat, dst)


def kernel(feats, coors, edge_index, params):
    n, d = feats.shape
    e = edge_index.shape[1]
    src = edge_index[0]
    dst = edge_index[1]
    c4 = jnp.pad(coors, ((0, 0), (0, 1)))

    p1 = params['e1']
    p4 = params['e4']
    lat = params['fc_w'].shape[1]
    m = p1['ew2'].shape[0]

    ws1, wd1, wdist1 = p1['ew1'][:d], p1['ew1'][d:2 * d], p1['ew1'][2 * d:2 * d + 1]
    ws2, wd2, wdist2 = p4['ew1'][:lat], p4['ew1'][lat:2 * lat], p4['ew1'][2 * lat:2 * lat + 1]
    eb1_1 = p1['eb1'].reshape(1, -1)
    eb1_2 = p4['eb1'].reshape(1, -1)
    cw2p = jnp.pad(p1['cw2'], ((0, 0), (0, 7)))
    cb2 = p1['cb2'].reshape(1, 1)
    bt = 4000

    nchunk = 4
    eh = e // nchunk
    srcs = tuple(src[i * eh:(i + 1) * eh] for i in range(nchunk))
    dsts = tuple(dst[i * eh:(i + 1) * eh] for i in range(nchunk))

    ps1, pd1 = _pre_call(feats, ws1, wd1, eb1_1)
    cpad = (-(n * 4)) % 128
    c4f1 = jnp.pad(c4.reshape(-1), (0, cpad))
    aggs1, css1 = [], []
    for sa, da in zip(srcs, dsts):
        g1, rd1f = _sc_gather(ps1, pd1, c4f1, sa, da)
        m1, rcw1 = _edge1_call(g1, rd1f.reshape(eh, 8), wdist1, p1['ew2'],
                               p1['eb2'].reshape(1, -1), p1['cw1'],
                               p1['cb1'].reshape(1, -1), cw2p, cb2, bt)
        aggs1.append(_sc_scatter_call(m1, da, n))
        css1.append(_sc_cscatter_call(rcw1.reshape(-1), da, n))
    npad = aggs1[0].shape[0] // 2
    nprt = 2 * nchunk
    aggp1 = jnp.concatenate(aggs1).reshape(nprt, npad, m)[:, :n]
    csp1 = jnp.concatenate(css1).reshape(nprt, npad, 128)[:, :n, 0:4]
    csp1 = jnp.transpose(csp1, (1, 0, 2)).reshape(n, 4 * nprt)
    f2, c24, ps2, pd2 = _node1_call(
        feats, c4, aggp1, csp1,
        p1['nw1'][:d], p1['nw1'][d:], p1['nb1'].reshape(1, -1),
        p1['nw2'], p1['nb2'].reshape(1, -1),
        params['fc_w'], params['fc_b'].reshape(1, -1),
        ws2, wd2, eb1_2)

    c4f2 = jnp.pad(c24.reshape(-1), (0, cpad))
    aggs2 = []
    for sa, da in zip(srcs, dsts):
        g2, rd2f = _sc_gather(ps2, pd2, c4f2, sa, da)
        m2 = _edge2_call(g2, rd2f.reshape(eh, 8), wdist2, p4['ew2'],
                         p4['eb2'].reshape(1, -1), bt)
        aggs2.append(_sc_scatter_call(m2, da, n))
    aggp2 = jnp.concatenate(aggs2).reshape(nprt, npad, m)[:, :n]
    out = _node2_call(
        f2, aggp2,
        p4['nw1'][:lat], p4['nw1'][lat:], p4['nb1'].reshape(1, -1),
        p4['nw2'], p4['nb2'].reshape(1, -1),
        params['fc1_w'], params['fc1_b'].reshape(1, -1))
    return out


# speedup vs baseline: 1.1065x; 1.1065x over previous
import functools

import jax
import jax.numpy as jnp
from jax import lax
from jax.experimental import pallas as pl
from jax.experimental.pallas import tpu as pltpu
from jax.experimental.pallas import tpu_sc as plsc

F32 = jnp.float32
I32 = jnp.int32

_NC = 2
_NS = 16
_NW = _NC * _NS
_B = 128


def _silu(x):
    return x * jax.nn.sigmoid(x)


def _pre_body(f_ref, ws_ref, wd_ref, eb_ref, ps_ref, pd_ref):
    f = f_ref[...]
    ps_ref[...] = jnp.dot(f, ws_ref[...], preferred_element_type=F32)
    pd_ref[...] = jnp.dot(f, wd_ref[...], preferred_element_type=F32) + eb_ref[...]


def _pre_call(f, ws, wd, eb):
    n = f.shape[0]
    m = ws.shape[1]
    return pl.pallas_call(
        _pre_body,
        out_shape=(jax.ShapeDtypeStruct((n, m), F32),
                   jax.ShapeDtypeStruct((n, m), F32)),
    )(f, ws, wd, eb)


def _edge1_body(g_ref, rd_ref, wdist_ref, ew2_ref, eb2_ref, cw1_ref, cb1_ref,
                cw2_ref, cb2_ref, mc_ref, rcw_ref):
    g = g_ref[...]
    rd = rd_ref[...]
    dist = rd[:, 3:4]
    h1 = _silu(g + dist * wdist_ref[...])
    m = _silu(jnp.dot(h1, ew2_ref[...], preferred_element_type=F32) + eb2_ref[...])
    t = _silu(jnp.dot(m, cw1_ref[...], preferred_element_type=F32) + cb1_ref[...])
    cw = jnp.dot(t, cw2_ref[...], preferred_element_type=F32)[:, 0:1] + cb2_ref[...]
    mask = (lax.broadcasted_iota(I32, rd.shape, 1) < 3).astype(F32)
    mc_ref[...] = m
    rcw_ref[...] = rd * cw * mask


def _edge1_call(g, rd, wdist, ew2, eb2, cw1, cb1, cw2p, cb2, bt):
    e = g.shape[0]
    mdim = g.shape[1]
    grid = e // bt
    full = lambda i: (0, 0)
    return pl.pallas_call(
        _edge1_body,
        grid=(grid,),
        in_specs=[
            pl.BlockSpec((bt, mdim), lambda i: (i, 0)),
            pl.BlockSpec((bt, 8), lambda i: (i, 0)),
            pl.BlockSpec(wdist.shape, full),
            pl.BlockSpec(ew2.shape, full),
            pl.BlockSpec(eb2.shape, full),
            pl.BlockSpec(cw1.shape, full),
            pl.BlockSpec(cb1.shape, full),
            pl.BlockSpec(cw2p.shape, full),
            pl.BlockSpec(cb2.shape, full),
        ],
        out_specs=(pl.BlockSpec((bt, mdim), lambda i: (i, 0)),
                   pl.BlockSpec((bt, 8), lambda i: (i, 0))),
        out_shape=(jax.ShapeDtypeStruct((e, mdim), F32),
                   jax.ShapeDtypeStruct((e, 8), F32)),
    )(g, rd, wdist, ew2, eb2, cw1, cb1, cw2p, cb2)


def _edge2_body(g_ref, rd_ref, wdist_ref, ew2_ref, eb2_ref, m_ref):
    g = g_ref[...]
    dist = rd_ref[...][:, 3:4]
    h1 = _silu(g + dist * wdist_ref[...])
    m_ref[...] = _silu(jnp.dot(h1, ew2_ref[...], preferred_element_type=F32)
                       + eb2_ref[...])


def _edge2_call(g, rd, wdist, ew2, eb2, bt):
    e = g.shape[0]
    mdim = g.shape[1]
    grid = e // bt
    full = lambda i: (0, 0)
    return pl.pallas_call(
        _edge2_body,
        grid=(grid,),
        in_specs=[
            pl.BlockSpec((bt, mdim), lambda i: (i, 0)),
            pl.BlockSpec((bt, 8), lambda i: (i, 0)),
            pl.BlockSpec(wdist.shape, full),
            pl.BlockSpec(ew2.shape, full),
            pl.BlockSpec(eb2.shape, full),
        ],
        out_specs=pl.BlockSpec((bt, mdim), lambda i: (i, 0)),
        out_shape=jax.ShapeDtypeStruct((e, mdim), F32),
    )(g, rd, wdist, ew2, eb2)


def _node1_body(f_ref, c4_ref, aggp_ref, csp_ref, nw1a_ref, nw1b_ref, nb1_ref,
                nw2_ref, nb2_ref, fcw_ref, fcb_ref, ws2_ref, wd2_ref, eb2l_ref,
                f2_ref, c24_ref, ps2_ref, pd2_ref):
    f = f_ref[...]
    ap = aggp_ref[...]
    agg = ap[0]
    for i in range(1, ap.shape[0]):
        agg = agg + ap[i]
    h = _silu(jnp.dot(f, nw1a_ref[...], preferred_element_type=F32)
              + jnp.dot(agg, nw1b_ref[...], preferred_element_type=F32)
              + nb1_ref[...])
    f1 = f + jnp.dot(h, nw2_ref[...], preferred_element_type=F32) + nb2_ref[...]
    f2 = jnp.dot(f1, fcw_ref[...], preferred_element_type=F32) + fcb_ref[...]
    f2_ref[...] = f2
    cp = csp_ref[...]
    cs = cp[:, 0:4]
    for i in range(1, cp.shape[1] // 4):
        cs = cs + cp[:, 4 * i:4 * i + 4]
    c24_ref[...] = c4_ref[...] + cs
    ps2_ref[...] = jnp.dot(f2, ws2_ref[...], preferred_element_type=F32)
    pd2_ref[...] = jnp.dot(f2, wd2_ref[...], preferred_element_type=F32) + eb2l_ref[...]


def _node1_call(f, c4, aggp, csp, nw1a, nw1b, nb1, nw2, nb2, fcw, fcb,
                ws2, wd2, eb2l):
    n, d = f.shape
    lat = fcw.shape[1]
    m = ws2.shape[1]
    np_ = aggp.shape[0]
    bt2 = 2000
    grid = n // bt2
    full = lambda i: (0, 0)
    return pl.pallas_call(
        _node1_body,
        grid=(grid,),
        in_specs=[
            pl.BlockSpec((bt2, d), lambda i: (i, 0)),
            pl.BlockSpec((bt2, 4), lambda i: (i, 0)),
            pl.BlockSpec((np_, bt2, m), lambda i: (0, i, 0)),
            pl.BlockSpec((bt2, csp.shape[1]), lambda i: (i, 0)),
            pl.BlockSpec(nw1a.shape, full),
            pl.BlockSpec(nw1b.shape, full),
            pl.BlockSpec(nb1.shape, full),
            pl.BlockSpec(nw2.shape, full),
            pl.BlockSpec(nb2.shape, full),
            pl.BlockSpec(fcw.shape, full),
            pl.BlockSpec(fcb.shape, full),
            pl.BlockSpec(ws2.shape, full),
            pl.BlockSpec(wd2.shape, full),
            pl.BlockSpec(eb2l.shape, full),
        ],
        out_specs=(pl.BlockSpec((bt2, lat), lambda i: (i, 0)),
                   pl.BlockSpec((bt2, 4), lambda i: (i, 0)),
                   pl.BlockSpec((bt2, m), lambda i: (i, 0)),
                   pl.BlockSpec((bt2, m), lambda i: (i, 0))),
        out_shape=(jax.ShapeDtypeStruct((n, lat), F32),
                   jax.ShapeDtypeStruct((n, 4), F32),
                   jax.ShapeDtypeStruct((n, m), F32),
                   jax.ShapeDtypeStruct((n, m), F32)),
    )(f, c4, aggp, csp, nw1a, nw1b, nb1, nw2, nb2, fcw, fcb, ws2, wd2, eb2l)


def _node2_body(f2_ref, aggp_ref, nw1a_ref, nw1b_ref, nb1_ref, nw2_ref,
                nb2_ref, fc1w_ref, fc1b_ref, out_ref):
    f2 = f2_ref[...]
    ap = aggp_ref[...]
    agg = ap[0]
    for i in range(1, ap.shape[0]):
        agg = agg + ap[i]
    h = _silu(jnp.dot(f2, nw1a_ref[...], preferred_element_type=F32)
              + jnp.dot(agg, nw1b_ref[...], preferred_element_type=F32)
              + nb1_ref[...])
    f3 = f2 + jnp.dot(h, nw2_ref[...], preferred_element_type=F32) + nb2_ref[...]
    out_ref[...] = jnp.dot(f3, fc1w_ref[...], preferred_element_type=F32) + fc1b_ref[...]


def _node2_call(f2, aggp, nw1a, nw1b, nb1, nw2, nb2, fc1w, fc1b):
    n, lat = f2.shape
    out = fc1w.shape[1]
    m = nw1b.shape[0]
    np_ = aggp.shape[0]
    bt2 = 2000
    grid = n // bt2
    full = lambda i: (0, 0)
    return pl.pallas_call(
        _node2_body,
        grid=(grid,),
        in_specs=[
            pl.BlockSpec((bt2, lat), lambda i: (i, 0)),
            pl.BlockSpec((np_, bt2, m), lambda i: (0, i, 0)),
            pl.BlockSpec(nw1a.shape, full),
            pl.BlockSpec(nw1b.shape, full),
            pl.BlockSpec(nb1.shape, full),
            pl.BlockSpec(nw2.shape, full),
            pl.BlockSpec(nb2.shape, full),
            pl.BlockSpec(fc1w.shape, full),
            pl.BlockSpec(fc1b.shape, full),
        ],
        out_specs=pl.BlockSpec((bt2, out), lambda i: (i, 0)),
        out_shape=jax.ShapeDtypeStruct((n, out), F32),
    )(f2, aggp, nw1a, nw1b, nb1, nw2, nb2, fc1w, fc1b)


def _sc_gather(ps, pd, c4flat, src, dst):
    n, m = ps.shape
    e = src.shape[0]
    nblk = e // _B
    mesh = plsc.VectorSubcoreMesh(core_axis_name="c", subcore_axis_name="s")
    grp = m // 16
    clen = c4flat.shape[0]

    @functools.partial(
        pl.kernel,
        out_type=(jax.ShapeDtypeStruct((e, m), F32),
                  jax.ShapeDtypeStruct((e * 8,), F32)),
        mesh=mesh,
        compiler_params=pltpu.CompilerParams(needs_layout_passes=False),
        scratch_types=[
            pltpu.VMEM((_B,), I32),
            pltpu.VMEM((_B,), I32),
            pltpu.VMEM((_B,), I32),
            pltpu.VMEM((_B,), I32),
            pltpu.VMEM((_B, m), F32),
            pltpu.VMEM((_B, m), F32),
            pltpu.VMEM((_B, m), F32),
            pltpu.VMEM((_B, m), F32),
            pltpu.VMEM((_B * 8,), F32),
            pltpu.VMEM((_B * 8,), F32),
            pltpu.VMEM((clen,), F32),
            pltpu.SemaphoreType.DMA,
            pltpu.SemaphoreType.DMA,
            pltpu.SemaphoreType.DMA,
            pltpu.SemaphoreType.DMA,
            pltpu.SemaphoreType.DMA,
            pltpu.SemaphoreType.DMA,
        ],
    )
    def gather_k(ps_hbm, pd_hbm, c4_hbm, src_hbm, dst_hbm, g_hbm, rd_hbm,
                 is0, id0, is1, id1, ba0, bb0, ba1, bb1, rb0, rb1, ctab,
                 sa0, sb0, sa1, sb1, sw0, sw1):
        wid = lax.axis_index("s") * _NC + lax.axis_index("c")
        pltpu.sync_copy(c4_hbm, ctab)
        zeros16 = jnp.zeros((16,), F32)

        def zbody(k, _):
            rb0[pl.ds(k * 16, 16)] = zeros16
            rb1[pl.ds(k * 16, 16)] = zeros16
            return 0
        lax.fori_loop(0, (_B * 8) // 16, zbody, 0)

        nmine = (nblk - 1 - wid) // _NW + 1
        npair = (nmine + 1) // 2
        iota16 = lax.iota(I32, 16)

        def reldist(isx, idxx, rb):
            for j in range(_B // 16):
                s16 = isx[pl.ds(j * 16, 16)] * 4
                d16 = idxx[pl.ds(j * 16, 16)] * 4
                xs = plsc.load_gather(ctab, [s16])
                ys = plsc.load_gather(ctab, [s16 + 1])
                zs = plsc.load_gather(ctab, [s16 + 2])
                xd = plsc.load_gather(ctab, [d16])
                yd = plsc.load_gather(ctab, [d16 + 1])
                zd = plsc.load_gather(ctab, [d16 + 2])
                rx = xs - xd
                ry = ys - yd
                rz = zs - zd
                dist = rx * rx + ry * ry + rz * rz
                off = (j * 16 + iota16) * 8
                plsc.store_scatter(rb, [off], rx)
                plsc.store_scatter(rb, [off + 1], ry)
                plsc.store_scatter(rb, [off + 2], rz)
                plsc.store_scatter(rb, [off + 3], dist)

        def addrows(ba, bb):
            def addrow(r, _):
                for k in range(grp):
                    sl = pl.ds(k * 16, 16)
                    plsc.addupdate(ba.at[r, sl], bb[r, sl])
                return 0
            lax.fori_loop(0, _B, addrow, 0)

        def pair_body(it, _):
            ja = it * 2
            jb = ja + 1
            hasb = jb < nmine
            basea = (wid + ja * _NW) * _B

            @pl.when(it > 0)
            def _():
                pltpu.make_async_copy(g_hbm.at[pl.ds(0, _B)], ba0, sw0).wait()
                pltpu.make_async_copy(rd_hbm.at[pl.ds(0, _B * 8)], rb0, sw0).wait()
                pltpu.make_async_copy(g_hbm.at[pl.ds(0, _B)], ba1, sw1).wait()
                pltpu.make_async_copy(rd_hbm.at[pl.ds(0, _B * 8)], rb1, sw1).wait()

            pltpu.sync_copy(src_hbm.at[pl.ds(basea, _B)], is0)
            pltpu.sync_copy(dst_hbm.at[pl.ds(basea, _B)], id0)
            cpa = pltpu.async_copy(ps_hbm.at[is0], ba0, sa0)
            cpb = pltpu.async_copy(pd_hbm.at[id0], bb0, sb0)

            @pl.when(hasb)
            def _():
                baseb = (wid + jb * _NW) * _B
                pltpu.sync_copy(src_hbm.at[pl.ds(baseb, _B)], is1)
                pltpu.sync_copy(dst_hbm.at[pl.ds(baseb, _B)], id1)
                pltpu.async_copy(ps_hbm.at[is1], ba1, sa1)
                pltpu.async_copy(pd_hbm.at[id1], bb1, sb1)

            reldist(is0, id0, rb0)
            cpa.wait()
            cpb.wait()
            addrows(ba0, bb0)
            pltpu.async_copy(ba0, g_hbm.at[pl.ds(basea, _B)], sw0)
            pltpu.async_copy(rb0, rd_hbm.at[pl.ds(basea * 8, _B * 8)], sw0)

            @pl.when(hasb)
            def _():
                baseb = (wid + jb * _NW) * _B
                reldist(is1, id1, rb1)
                pltpu.make_async_copy(ps_hbm.at[pl.ds(0, _B)], ba1, sa1).wait()
                pltpu.make_async_copy(pd_hbm.at[pl.ds(0, _B)], bb1, sb1).wait()
                addrows(ba1, bb1)
                pltpu.async_copy(ba1, g_hbm.at[pl.ds(baseb, _B)], sw1)
                pltpu.async_copy(rb1, rd_hbm.at[pl.ds(baseb * 8, _B * 8)], sw1)
            return 0

        lax.fori_loop(0, npair, pair_body, 0)

        pltpu.make_async_copy(g_hbm.at[pl.ds(0, _B)], ba0, sw0).wait()
        pltpu.make_async_copy(rd_hbm.at[pl.ds(0, _B * 8)], rb0, sw0).wait()

        @pl.when((nmine % 2) == 0)
        def _():
            pltpu.make_async_copy(g_hbm.at[pl.ds(0, _B)], ba1, sw1).wait()
            pltpu.make_async_copy(rd_hbm.at[pl.ds(0, _B * 8)], rb1, sw1).wait()

    return gather_k(ps, pd, c4flat, src, dst)


def _sc_scatter_call(mrows, dst, n):
    e, mdim = mrows.shape
    nblk = e // _B
    zrows = 16
    npad = -(-n // (_NS * zrows)) * (_NS * zrows)
    rows_per_tile = npad // _NS
    nz = rows_per_tile // zrows
    mesh = plsc.VectorSubcoreMesh(core_axis_name="c", subcore_axis_name="s")
    grp = mdim // 16

    @functools.partial(
        pl.kernel,
        out_type=jax.ShapeDtypeStruct((2 * npad, mdim), F32),
        mesh=mesh,
        compiler_params=pltpu.CompilerParams(needs_layout_passes=False),
        scratch_types=[
            pltpu.VMEM((_B,), I32),
            pltpu.VMEM((_B,), I32),
            pltpu.VMEM((_B, mdim), F32),
            pltpu.VMEM((_B, mdim), F32),
            pltpu.VMEM((zrows, mdim), F32),
            pltpu.VMEM_SHARED((npad, mdim), F32),
            pltpu.SemaphoreType.DMA,
            pltpu.SemaphoreType.DMA,
            pltpu.SemaphoreType.DMA,
            pltpu.SemaphoreType.DMA,
        ],
    )
    def body(m_hbm, dst_hbm, aggp_hbm, ix0, ix1, mb0, mb1, zbuf, agg_sh,
             sl0, sl1, ss0, ss1):
        cid = lax.axis_index("c")
        sid = lax.axis_index("s")
        wid = sid * _NC + cid
        zeros16 = jnp.zeros((16,), F32)

        def zrow(r, _):
            for k in range(grp):
                zbuf[r, pl.ds(k * 16, 16)] = zeros16
            return 0
        lax.fori_loop(0, zrows, zrow, 0)

        base_row = sid * rows_per_tile
        for k in range(nz):
            pltpu.sync_copy(zbuf, agg_sh.at[pl.ds(base_row + k * zrows, zrows)])
        plsc.subcore_barrier()

        nmine = (nblk - 1 - wid) // _NW + 1
        npair = (nmine + 1) // 2

        def pair_body(it, _):
            ja = it * 2
            jb = ja + 1
            hasb = jb < nmine
            basea = (wid + ja * _NW) * _B

            @pl.when(it > 0)
            def _():
                pltpu.make_async_copy(m_hbm.at[pl.ds(0, _B)], mb0, ss0).wait()
                pltpu.make_async_copy(m_hbm.at[pl.ds(0, _B)], mb1, ss1).wait()

            cpi = pltpu.async_copy(dst_hbm.at[pl.ds(basea, _B)], ix0, sl0)
            cpm = pltpu.async_copy(m_hbm.at[pl.ds(basea, _B)], mb0, sl0)

            @pl.when(hasb)
            def _():
                baseb = (wid + jb * _NW) * _B
                pltpu.async_copy(dst_hbm.at[pl.ds(baseb, _B)], ix1, sl1)
                pltpu.async_copy(m_hbm.at[pl.ds(baseb, _B)], mb1, sl1)

            cpi.wait()
            cpm.wait()
            pltpu.async_copy(mb0, agg_sh.at[ix0], ss0, add=True)

            @pl.when(hasb)
            def _():
                pltpu.make_async_copy(dst_hbm.at[pl.ds(0, _B)], ix1, sl1).wait()
                pltpu.make_async_copy(m_hbm.at[pl.ds(0, _B)], mb1, sl1).wait()
                pltpu.async_copy(mb1, agg_sh.at[ix1], ss1, add=True)
            return 0

        lax.fori_loop(0, npair, pair_body, 0)

        pltpu.make_async_copy(m_hbm.at[pl.ds(0, _B)], mb0, ss0).wait()

        @pl.when((nmine % 2) == 0)
        def _():
            pltpu.make_async_copy(m_hbm.at[pl.ds(0, _B)], mb1, ss1).wait()
        plsc.subcore_barrier()

        out_base = cid * npad + base_row
        pltpu.sync_copy(agg_sh.at[pl.ds(base_row, rows_per_tile)],
                        aggp_hbm.at[pl.ds(out_base, rows_per_tile)])

    return body(mrows, dst)


def _sc_cscatter_call(rcwflat, dst, n):
    e = dst.shape[0]
    nblk = e // _B
    zrows = 16
    npad = -(-n // (_NS * zrows)) * (_NS * zrows)
    rows_per_tile = npad // _NS
    nz = rows_per_tile // zrows
    mesh = plsc.VectorSubcoreMesh(core_axis_name="c", subcore_axis_name="s")

    @functools.partial(
        pl.kernel,
        out_type=jax.ShapeDtypeStruct((2 * npad, 128), F32),
        mesh=mesh,
        compiler_params=pltpu.CompilerParams(needs_layout_passes=False),
        scratch_types=[
            pltpu.VMEM((_B,), I32),
            pltpu.VMEM((_B,), I32),
            pltpu.VMEM((_B * 8,), F32),
            pltpu.VMEM((_B * 8,), F32),
            pltpu.VMEM((_B, 128), F32),
            pltpu.VMEM((_B, 128), F32),
            pltpu.VMEM((zrows, 128), F32),
            pltpu.VMEM_SHARED((npad, 128), F32),
            pltpu.SemaphoreType.DMA,
            pltpu.SemaphoreType.DMA,
            pltpu.SemaphoreType.DMA,
            pltpu.SemaphoreType.DMA,
        ],
    )
    def body(rcw_hbm, dst_hbm, csp_hbm, ix0, ix1, rb0, rb1, eb0, eb1,
             zbuf, cs_sh, sl0, sl1, ss0, ss1):
        cid = lax.axis_index("c")
        sid = lax.axis_index("s")
        wid = sid * _NC + cid
        zeros16 = jnp.zeros((16,), F32)
        iota16 = lax.iota(I32, 16)

        def zrow(r, _):
            for k in range(8):
                zbuf[r, pl.ds(k * 16, 16)] = zeros16
            return 0
        lax.fori_loop(0, zrows, zrow, 0)

        def zerow(r, _):
            for k in range(8):
                eb0[r, pl.ds(k * 16, 16)] = zeros16
                eb1[r, pl.ds(k * 16, 16)] = zeros16
            return 0
        lax.fori_loop(0, _B, zerow, 0)
        base_row = sid * rows_per_tile
        for k in range(nz):
            pltpu.sync_copy(zbuf, cs_sh.at[pl.ds(base_row + k * zrows, zrows)])
        plsc.subcore_barrier()

        nmine = (nblk - 1 - wid) // _NW + 1
        npair = (nmine + 1) // 2

        def expand(rb, eb):
            for g in range(_B // 16):
                rows = g * 16 + iota16
                for c in range(3):
                    v = plsc.load_gather(rb, [rows * 8 + c])
                    plsc.store_scatter(eb, [rows, jnp.zeros((16,), I32) + c], v)

        def pair_body(it, _):
            ja = it * 2
            jb = ja + 1
            hasb = jb < nmine
            basea = (wid + ja * _NW) * _B

            @pl.when(it > 0)
            def _():
                pltpu.make_async_copy(csp_hbm.at[pl.ds(0, _B)], eb0, ss0).wait()
                pltpu.make_async_copy(csp_hbm.at[pl.ds(0, _B)], eb1, ss1).wait()

            cpi = pltpu.async_copy(dst_hbm.at[pl.ds(basea, _B)], ix0, sl0)
            cpr = pltpu.async_copy(rcw_hbm.at[pl.ds(basea * 8, _B * 8)], rb0, sl0)

            @pl.when(hasb)
            def _():
                baseb = (wid + jb * _NW) * _B
                pltpu.async_copy(dst_hbm.at[pl.ds(baseb, _B)], ix1, sl1)
                pltpu.async_copy(rcw_hbm.at[pl.ds(baseb * 8, _B * 8)], rb1, sl1)

            cpi.wait()
            cpr.wait()
            expand(rb0, eb0)
            pltpu.async_copy(eb0, cs_sh.at[ix0], ss0, add=True)

            @pl.when(hasb)
            def _():
                pltpu.make_async_copy(dst_hbm.at[pl.ds(0, _B)], ix1, sl1).wait()
                pltpu.make_async_copy(rcw_hbm.at[pl.ds(0, _B * 8)], rb1, sl1).wait()
                expand(rb1, eb1)
                pltpu.async_copy(eb1, cs_sh.at[ix1], ss1, add=True)
            return 0

        lax.fori_loop(0, npair, pair_body, 0)

        pltpu.make_async_copy(csp_hbm.at[pl.ds(0, _B)], eb0, ss0).wait()

        @pl.when((nmine % 2) == 0)
        def _():
            pltpu.make_async_copy(csp_hbm.at[pl.ds(0, _B)], eb1, ss1).wait()
        plsc.subcore_barrier()

        out_base = cid * npad + base_row
        pltpu.sync_copy(cs_sh.at[pl.ds(base_row, rows_per_tile)],
                        csp_hbm.at[pl.ds(out_base, rows_per_tile)])

    return body(rcwflat, dst)


def kernel(feats, coors, edge_index, params):
    n, d = feats.shape
    e = edge_index.shape[1]
    src = edge_index[0]
    dst = edge_index[1]
    c4 = jnp.pad(coors, ((0, 0), (0, 1)))

    p1 = params['e1']
    p4 = params['e4']
    lat = params['fc_w'].shape[1]
    m = p1['ew2'].shape[0]

    ws1, wd1, wdist1 = p1['ew1'][:d], p1['ew1'][d:2 * d], p1['ew1'][2 * d:2 * d + 1]
    ws2, wd2, wdist2 = p4['ew1'][:lat], p4['ew1'][lat:2 * lat], p4['ew1'][2 * lat:2 * lat + 1]
    eb1_1 = p1['eb1'].reshape(1, -1)
    eb1_2 = p4['eb1'].reshape(1, -1)
    cw2p = jnp.pad(p1['cw2'], ((0, 0), (0, 7)))
    cb2 = p1['cb2'].reshape(1, 1)
    bt = 8000

    eh = e // 2
    srcs = (src[:eh], src[eh:])
    dsts = (dst[:eh], dst[eh:])

    ps1, pd1 = _pre_call(feats, ws1, wd1, eb1_1)
    cpad = (-(n * 4)) % 128
    c4f1 = jnp.pad(c4.reshape(-1), (0, cpad))
    aggs1, css1 = [], []
    for sa, da in zip(srcs, dsts):
        g1, rd1f = _sc_gather(ps1, pd1, c4f1, sa, da)
        m1, rcw1 = _edge1_call(g1, rd1f.reshape(eh, 8), wdist1, p1['ew2'],
                               p1['eb2'].reshape(1, -1), p1['cw1'],
                               p1['cb1'].reshape(1, -1), cw2p, cb2, bt)
        aggs1.append(_sc_scatter_call(m1, da, n))
        css1.append(_sc_cscatter_call(rcw1.reshape(-1), da, n))
    npad = aggs1[0].shape[0] // 2
    aggp1 = jnp.concatenate(aggs1).reshape(4, npad, m)[:, :n]
    csp1 = jnp.concatenate(css1).reshape(4, npad, 128)[:, :n, 0:4]
    csp1 = jnp.transpose(csp1, (1, 0, 2)).reshape(n, 16)
    f2, c24, ps2, pd2 = _node1_call(
        feats, c4, aggp1, csp1,
        p1['nw1'][:d], p1['nw1'][d:], p1['nb1'].reshape(1, -1),
        p1['nw2'], p1['nb2'].reshape(1, -1),
        params['fc_w'], params['fc_b'].reshape(1, -1),
        ws2, wd2, eb1_2)

    c4f2 = jnp.pad(c24.reshape(-1), (0, cpad))
    aggs2 = []
    for sa, da in zip(srcs, dsts):
        g2, rd2f = _sc_gather(ps2, pd2, c4f2, sa, da)
        m2 = _edge2_call(g2, rd2f.reshape(eh, 8), wdist2, p4['ew2'],
                         p4['eb2'].reshape(1, -1), bt)
        aggs2.append(_sc_scatter_call(m2, da, n))
    aggp2 = jnp.concatenate(aggs2).reshape(4, npad, m)[:, :n]
    out = _node2_call(
        f2, aggp2,
        p4['nw1'][:lat], p4['nw1'][lat:], p4['nb1'].reshape(1, -1),
        p4['nw2'], p4['nb2'].reshape(1, -1),
        params['fc1_w'], params['fc1_b'].reshape(1, -1))
    return out
